# baseline (speedup 1.0000x reference)
import jax
import jax.numpy as jnp
from jax import lax
from jax.experimental import pallas as pl
from jax.experimental.pallas import tpu as pltpu
from jax.experimental.pallas import tpu_sc as plsc

N_NODES = 10000
N_EDGES = 320000
FEAT = 128
NUM_CORES = 2
NUM_SUBCORES = 16
NUM_TILES = NUM_CORES * NUM_SUBCORES
GROUP = 80
N_PAD = 10240
EDGES_PER_TILE = N_EDGES // NUM_TILES
GROUPS = EDGES_PER_TILE // GROUP
ROWS_PER_TILE = N_PAD // NUM_SUBCORES


def _sc_body(src_hbm, dst_hbm, state_hbm, out0, out1,
             idx_s, idx_d, rows, sem_i, sem_g, shared):
    cid = lax.axis_index("c")
    sid = lax.axis_index("s")
    wid = cid * NUM_SUBCORES + sid

    zeros16 = jnp.zeros((16,), jnp.float32)

    def _zrow(r, _):
        def _zcol(j, _):
            rows[0, r, pl.ds(j * 16, 16)] = zeros16
            return 0
        return lax.fori_loop(0, FEAT // 16, _zcol, 0)

    lax.fori_loop(0, GROUP, _zrow, 0)

    row0 = sid * ROWS_PER_TILE
    for b in range(ROWS_PER_TILE // GROUP):
        pltpu.sync_copy(rows.at[0], shared.at[pl.ds(row0 + b * GROUP, GROUP)])
    plsc.subcore_barrier()

    ebase = wid * EDGES_PER_TILE

    def _fire_idx(g, slot):
        off = ebase + g * GROUP
        pltpu.async_copy(src_hbm.at[pl.ds(off, GROUP)], idx_s.at[slot],
                         sem_i.at[slot])
        pltpu.async_copy(dst_hbm.at[pl.ds(off, GROUP)], idx_d.at[slot],
                         sem_i.at[slot])

    def _drain_idx(slot):
        pltpu.make_async_copy(src_hbm.at[pl.ds(0, GROUP)], idx_s.at[slot],
                              sem_i.at[slot]).wait()
        pltpu.make_async_copy(src_hbm.at[pl.ds(0, GROUP)], idx_d.at[slot],
                              sem_i.at[slot]).wait()

    def _drain_rows(rslot, sem):
        pltpu.make_async_copy(state_hbm.at[pl.ds(0, GROUP)], rows.at[rslot],
                              sem.at[rslot]).wait()

    def _fire_gather(islot, rslot):
        pltpu.async_copy(state_hbm.at[idx_s.at[islot]], rows.at[rslot],
                         sem_g.at[rslot])

    _fire_idx(0, 0)
    _fire_idx(1, 1)
    _fire_idx(2, 2)
    _drain_idx(0)
    _fire_gather(0, 0)

    def _iter(i, _):
        for j in range(4):
            g = i * 4 + j
            rslot = j % 2
            nslot = (j + 1) % 2

            @pl.when(g + 3 < GROUPS)
            def _():
                _fire_idx(g + 3, (j + 3) % 4)

            @pl.when(g + 1 < GROUPS)
            def _():
                _drain_idx((j + 1) % 4)
                _fire_gather((j + 1) % 4, nslot)

            _drain_rows(rslot, sem_g)
            pltpu.sync_copy(rows.at[rslot], shared.at[idx_d.at[j]], add=True)
        return 0

    lax.fori_loop(0, GROUPS // 4, _iter, 0)
    _drain_rows(0, sem_g)
    pltpu.sync_copy(rows.at[0], shared.at[idx_d.at[0]], add=True)
    plsc.subcore_barrier()

    @pl.when(cid == 0)
    def _():
        pltpu.sync_copy(shared.at[pl.ds(row0, ROWS_PER_TILE)],
                        out0.at[pl.ds(row0, ROWS_PER_TILE)])

    @pl.when(cid == 1)
    def _():
        pltpu.sync_copy(shared.at[pl.ds(row0, ROWS_PER_TILE)],
                        out1.at[pl.ds(row0, ROWS_PER_TILE)])


@jax.jit
def _sc_scatter(src, dst, state):
    mesh = plsc.VectorSubcoreMesh(core_axis_name="c", subcore_axis_name="s")
    f = pl.kernel(
        _sc_body,
        out_type=[jax.ShapeDtypeStruct((N_PAD, FEAT), jnp.float32),
                  jax.ShapeDtypeStruct((N_PAD, FEAT), jnp.float32)],
        mesh=mesh,
        scratch_types=[
            pltpu.VMEM((4, GROUP), jnp.int32),
            pltpu.VMEM((4, GROUP), jnp.int32),
            pltpu.VMEM((2, GROUP, FEAT), jnp.float32),
            pltpu.SemaphoreType.DMA((4,)),
            pltpu.SemaphoreType.DMA((2,)),
            pltpu.VMEM_SHARED((N_PAD, FEAT), jnp.float32),
        ],
    )
    return f(src, dst, state)


def _tc_in_body(x_ref, win_ref, o_ref):
    dn = (((1,), (1,)), ((), ()))
    o_ref[...] = lax.dot_general(x_ref[...], win_ref[...], dn,
                                 preferred_element_type=jnp.float32)


def _tc_out_body(leak_ref, pin_ref, s_ref, p0_ref, p1_ref, wrec_ref, o_ref):
    aggr = p0_ref[...] + p1_ref[...]
    dn = (((1,), (1,)), ((), ()))
    pre = pin_ref[...] + lax.dot_general(aggr, wrec_ref[...], dn,
                                         preferred_element_type=jnp.float32)
    lam = leak_ref[0, 0]
    o_ref[...] = lam * jnp.tanh(pre) + (1.0 - lam) * s_ref[...]


_BLK = 1000
_ROW_SPEC = pl.BlockSpec((_BLK, FEAT), lambda i: (i, 0))
_W_SPEC = pl.BlockSpec((FEAT, FEAT), lambda i: (0, 0))


@jax.jit
def _tc_in(x, W_in):
    return pl.pallas_call(
        _tc_in_body,
        grid=(N_NODES // _BLK,),
        in_specs=[_ROW_SPEC, _W_SPEC],
        out_specs=_ROW_SPEC,
        out_shape=jax.ShapeDtypeStruct((N_NODES, FEAT), jnp.float32),
    )(x, W_in)


@jax.jit
def _tc_out(leak, pin, s, p0, p1, W_rec):
    return pl.pallas_call(
        _tc_out_body,
        grid=(N_NODES // _BLK,),
        in_specs=[
            pl.BlockSpec(memory_space=pltpu.SMEM),
            _ROW_SPEC, _ROW_SPEC, _ROW_SPEC, _ROW_SPEC, _W_SPEC,
        ],
        out_specs=_ROW_SPEC,
        out_shape=jax.ShapeDtypeStruct((N_NODES, FEAT), jnp.float32),
    )(leak, pin, s, p0, p1, W_rec)


def kernel(edge_index, input, state, W_in, W_rec, leakage):
    src = edge_index[0].astype(jnp.int32)
    dst = edge_index[1].astype(jnp.int32)
    p0, p1 = _sc_scatter(src, dst, state)
    pin = _tc_in(input, W_in)
    leak2d = jnp.asarray(leakage, jnp.float32).reshape(1, 1)
    return _tc_out(leak2d, pin, state, p0, p1, W_rec)


# speedup vs baseline: 1.1329x; 1.1329x over previous
import jax
import jax.numpy as jnp
from jax import lax
from jax.experimental import pallas as pl
from jax.experimental.pallas import tpu as pltpu
from jax.experimental.pallas import tpu_sc as plsc

N_NODES = 10000
N_EDGES = 320000
FEAT = 128
NUM_CORES = 2
NUM_SUBCORES = 16
NUM_TILES = NUM_CORES * NUM_SUBCORES
GROUP = 40
N_PAD = 10240
EDGES_PER_TILE = N_EDGES // NUM_TILES
GROUPS = EDGES_PER_TILE // GROUP
ROWS_PER_TILE = N_PAD // NUM_SUBCORES
NROW_SLOTS = 4
NIDX_SLOTS = 8


def _sc_body(src_hbm, dst_hbm, state_hbm, out0, out1,
             idx_s, idx_d, rows, sem_i, sem_g, shared):
    cid = lax.axis_index("c")
    sid = lax.axis_index("s")
    wid = cid * NUM_SUBCORES + sid

    zeros16 = jnp.zeros((16,), jnp.float32)

    def _zrow(r, _):
        def _zcol(j, _):
            rows[0, r, pl.ds(j * 16, 16)] = zeros16
            return 0
        return lax.fori_loop(0, FEAT // 16, _zcol, 0)

    lax.fori_loop(0, GROUP, _zrow, 0)

    row0 = sid * ROWS_PER_TILE
    for b in range(ROWS_PER_TILE // GROUP):
        pltpu.sync_copy(rows.at[0], shared.at[pl.ds(row0 + b * GROUP, GROUP)])
    plsc.subcore_barrier()

    ebase = wid * EDGES_PER_TILE

    def _fire_idx(g, slot):
        off = ebase + g * GROUP
        pltpu.async_copy(src_hbm.at[pl.ds(off, GROUP)], idx_s.at[slot],
                         sem_i.at[slot])
        pltpu.async_copy(dst_hbm.at[pl.ds(off, GROUP)], idx_d.at[slot],
                         sem_i.at[slot])

    def _drain_idx(slot):
        pltpu.make_async_copy(src_hbm.at[pl.ds(0, GROUP)], idx_s.at[slot],
                              sem_i.at[slot]).wait()
        pltpu.make_async_copy(src_hbm.at[pl.ds(0, GROUP)], idx_d.at[slot],
                              sem_i.at[slot]).wait()

    def _drain_rows(rslot, sem):
        pltpu.make_async_copy(state_hbm.at[pl.ds(0, GROUP)], rows.at[rslot],
                              sem.at[rslot]).wait()

    def _fire_gather(islot, rslot):
        pltpu.async_copy(state_hbm.at[idx_s.at[islot]], rows.at[rslot],
                         sem_g.at[rslot])

    for g0 in range(NIDX_SLOTS - 1):
        _fire_idx(g0, g0)
    for g0 in range(NROW_SLOTS - 1):
        _drain_idx(g0)
        _fire_gather(g0, g0)

    def _iter(i, _):
        for j in range(NIDX_SLOTS):
            g = i * NIDX_SLOTS + j
            rslot = j % NROW_SLOTS

            @pl.when(g + NIDX_SLOTS - 1 < GROUPS)
            def _():
                _fire_idx(g + NIDX_SLOTS - 1, (j + NIDX_SLOTS - 1) % NIDX_SLOTS)

            @pl.when(g + NROW_SLOTS - 1 < GROUPS)
            def _():
                _drain_idx((j + NROW_SLOTS - 1) % NIDX_SLOTS)
                _fire_gather((j + NROW_SLOTS - 1) % NIDX_SLOTS,
                             (j + NROW_SLOTS - 1) % NROW_SLOTS)

            _drain_rows(rslot, sem_g)
            pltpu.sync_copy(rows.at[rslot], shared.at[idx_d.at[j]], add=True)
        return 0

    lax.fori_loop(0, GROUPS // NIDX_SLOTS, _iter, 0)
    for t in range(GROUPS - (GROUPS // NIDX_SLOTS) * NIDX_SLOTS):
        _drain_rows(t % NROW_SLOTS, sem_g)
        pltpu.sync_copy(rows.at[t % NROW_SLOTS], shared.at[idx_d.at[t]],
                        add=True)
    plsc.subcore_barrier()

    @pl.when(cid == 0)
    def _():
        pltpu.sync_copy(shared.at[pl.ds(row0, ROWS_PER_TILE)],
                        out0.at[pl.ds(row0, ROWS_PER_TILE)])

    @pl.when(cid == 1)
    def _():
        pltpu.sync_copy(shared.at[pl.ds(row0, ROWS_PER_TILE)],
                        out1.at[pl.ds(row0, ROWS_PER_TILE)])


@jax.jit
def _sc_scatter(src, dst, state):
    mesh = plsc.VectorSubcoreMesh(core_axis_name="c", subcore_axis_name="s")
    f = pl.kernel(
        _sc_body,
        out_type=[jax.ShapeDtypeStruct((N_PAD, FEAT), jnp.float32),
                  jax.ShapeDtypeStruct((N_PAD, FEAT), jnp.float32)],
        mesh=mesh,
        scratch_types=[
            pltpu.VMEM((NIDX_SLOTS, GROUP), jnp.int32),
            pltpu.VMEM((NIDX_SLOTS, GROUP), jnp.int32),
            pltpu.VMEM((NROW_SLOTS, GROUP, FEAT), jnp.float32),
            pltpu.SemaphoreType.DMA((NIDX_SLOTS,)),
            pltpu.SemaphoreType.DMA((NROW_SLOTS,)),
            pltpu.VMEM_SHARED((N_PAD, FEAT), jnp.float32),
        ],
    )
    return f(src, dst, state)


def _tc_in_body(x_ref, win_ref, o_ref):
    dn = (((1,), (1,)), ((), ()))
    o_ref[...] = lax.dot_general(x_ref[...], win_ref[...], dn,
                                 preferred_element_type=jnp.float32)


def _tc_out_body(leak_ref, pin_ref, s_ref, p0_ref, p1_ref, wrec_ref, o_ref):
    aggr = p0_ref[...] + p1_ref[...]
    dn = (((1,), (1,)), ((), ()))
    pre = pin_ref[...] + lax.dot_general(aggr, wrec_ref[...], dn,
                                         preferred_element_type=jnp.float32)
    lam = leak_ref[0, 0]
    o_ref[...] = lam * jnp.tanh(pre) + (1.0 - lam) * s_ref[...]


_BLK = 1000
_ROW_SPEC = pl.BlockSpec((_BLK, FEAT), lambda i: (i, 0))
_W_SPEC = pl.BlockSpec((FEAT, FEAT), lambda i: (0, 0))


@jax.jit
def _tc_in(x, W_in):
    return pl.pallas_call(
        _tc_in_body,
        grid=(N_NODES // _BLK,),
        in_specs=[_ROW_SPEC, _W_SPEC],
        out_specs=_ROW_SPEC,
        out_shape=jax.ShapeDtypeStruct((N_NODES, FEAT), jnp.float32),
    )(x, W_in)


@jax.jit
def _tc_out(leak, pin, s, p0, p1, W_rec):
    return pl.pallas_call(
        _tc_out_body,
        grid=(N_NODES // _BLK,),
        in_specs=[
            pl.BlockSpec(memory_space=pltpu.SMEM),
            _ROW_SPEC, _ROW_SPEC, _ROW_SPEC, _ROW_SPEC, _W_SPEC,
        ],
        out_specs=_ROW_SPEC,
        out_shape=jax.ShapeDtypeStruct((N_NODES, FEAT), jnp.float32),
    )(leak, pin, s, p0, p1, W_rec)


def kernel(edge_index, input, state, W_in, W_rec, leakage):
    src = edge_index[0].astype(jnp.int32)
    dst = edge_index[1].astype(jnp.int32)
    p0, p1 = _sc_scatter(src, dst, state)
    pin = _tc_in(input, W_in)
    leak2d = jnp.asarray(leakage, jnp.float32).reshape(1, 1)
    return _tc_out(leak2d, pin, state, p0, p1, W_rec)


# speedup vs baseline: 1.1329x; 1.0000x over previous
import jax
import jax.numpy as jnp
from jax import lax
from jax.experimental import pallas as pl
from jax.experimental.pallas import tpu as pltpu
from jax.experimental.pallas import tpu_sc as plsc

N_NODES = 10000
N_EDGES = 320000
FEAT = 128
NUM_CORES = 2
NUM_SUBCORES = 16
NUM_TILES = NUM_CORES * NUM_SUBCORES
GROUP = 40
N_PAD = 10240
EDGES_PER_TILE = N_EDGES // NUM_TILES
GROUPS = EDGES_PER_TILE // GROUP
ROWS_PER_TILE = N_PAD // NUM_SUBCORES
NROW_SLOTS = 4
NIDX_SLOTS = 8


def _sc_body(src_hbm, dst_hbm, state_hbm, out0, out1,
             idx_s, idx_d, rows, sem_i, sem_g, sem_s, shared):
    cid = lax.axis_index("c")
    sid = lax.axis_index("s")
    wid = cid * NUM_SUBCORES + sid

    zeros16 = jnp.zeros((16,), jnp.float32)

    def _zrow(r, _):
        def _zcol(j, _):
            rows[0, r, pl.ds(j * 16, 16)] = zeros16
            return 0
        return lax.fori_loop(0, FEAT // 16, _zcol, 0)

    lax.fori_loop(0, GROUP, _zrow, 0)

    row0 = sid * ROWS_PER_TILE
    for b in range(ROWS_PER_TILE // GROUP):
        pltpu.sync_copy(rows.at[0], shared.at[pl.ds(row0 + b * GROUP, GROUP)])
    plsc.subcore_barrier()

    ebase = wid * EDGES_PER_TILE

    def _fire_idx(g, slot):
        off = ebase + g * GROUP
        pltpu.async_copy(src_hbm.at[pl.ds(off, GROUP)], idx_s.at[slot],
                         sem_i.at[slot])
        pltpu.async_copy(dst_hbm.at[pl.ds(off, GROUP)], idx_d.at[slot],
                         sem_i.at[slot])

    def _drain_idx(slot):
        pltpu.make_async_copy(src_hbm.at[pl.ds(0, GROUP)], idx_s.at[slot],
                              sem_i.at[slot]).wait()
        pltpu.make_async_copy(src_hbm.at[pl.ds(0, GROUP)], idx_d.at[slot],
                              sem_i.at[slot]).wait()

    def _drain_rows(rslot, sem):
        pltpu.make_async_copy(state_hbm.at[pl.ds(0, GROUP)], rows.at[rslot],
                              sem.at[rslot]).wait()

    def _fire_gather(islot, rslot):
        pltpu.async_copy(state_hbm.at[idx_s.at[islot]], rows.at[rslot],
                         sem_g.at[rslot])

    for g0 in range(NIDX_SLOTS - 1):
        _fire_idx(g0, g0)
    for g0 in range(NROW_SLOTS - 1):
        _drain_idx(g0)
        _fire_gather(g0, g0)

    def _iter(i, _):
        for j in range(NIDX_SLOTS):
            g = i * NIDX_SLOTS + j
            rslot = j % NROW_SLOTS
            fslot = (j + NROW_SLOTS - 1) % NROW_SLOTS

            @pl.when(g + NROW_SLOTS - 1 < GROUPS)
            def _():
                @pl.when(g >= 1)
                def _():
                    _drain_rows(fslot, sem_s)
                _drain_idx((j + NROW_SLOTS - 1) % NIDX_SLOTS)
                _fire_gather((j + NROW_SLOTS - 1) % NIDX_SLOTS, fslot)

            @pl.when(g + NIDX_SLOTS - 1 < GROUPS)
            def _():
                _fire_idx(g + NIDX_SLOTS - 1, (j + NIDX_SLOTS - 1) % NIDX_SLOTS)

            _drain_rows(rslot, sem_g)
            pltpu.async_copy(rows.at[rslot], shared.at[idx_d.at[j]],
                             sem_s.at[rslot], add=True)
        return 0

    lax.fori_loop(0, GROUPS // NIDX_SLOTS, _iter, 0)
    ntail = GROUPS - (GROUPS // NIDX_SLOTS) * NIDX_SLOTS
    for t in range(ntail):
        _drain_rows(t % NROW_SLOTS, sem_g)
        pltpu.async_copy(rows.at[t % NROW_SLOTS], shared.at[idx_d.at[t]],
                         sem_s.at[t % NROW_SLOTS], add=True)
    for t in range(NROW_SLOTS):
        _drain_rows((ntail + t) % NROW_SLOTS, sem_s)
    plsc.subcore_barrier()

    @pl.when(cid == 0)
    def _():
        pltpu.sync_copy(shared.at[pl.ds(row0, ROWS_PER_TILE)],
                        out0.at[pl.ds(row0, ROWS_PER_TILE)])

    @pl.when(cid == 1)
    def _():
        pltpu.sync_copy(shared.at[pl.ds(row0, ROWS_PER_TILE)],
                        out1.at[pl.ds(row0, ROWS_PER_TILE)])


@jax.jit
def _sc_scatter(src, dst, state):
    mesh = plsc.VectorSubcoreMesh(core_axis_name="c", subcore_axis_name="s")
    f = pl.kernel(
        _sc_body,
        out_type=[jax.ShapeDtypeStruct((N_PAD, FEAT), jnp.float32),
                  jax.ShapeDtypeStruct((N_PAD, FEAT), jnp.float32)],
        mesh=mesh,
        scratch_types=[
            pltpu.VMEM((NIDX_SLOTS, GROUP), jnp.int32),
            pltpu.VMEM((NIDX_SLOTS, GROUP), jnp.int32),
            pltpu.VMEM((NROW_SLOTS, GROUP, FEAT), jnp.float32),
            pltpu.SemaphoreType.DMA((NIDX_SLOTS,)),
            pltpu.SemaphoreType.DMA((NROW_SLOTS,)),
            pltpu.SemaphoreType.DMA((NROW_SLOTS,)),
            pltpu.VMEM_SHARED((N_PAD, FEAT), jnp.float32),
        ],
    )
    return f(src, dst, state)


def _tc_in_body(x_ref, win_ref, o_ref):
    dn = (((1,), (1,)), ((), ()))
    o_ref[...] = lax.dot_general(x_ref[...], win_ref[...], dn,
                                 preferred_element_type=jnp.float32)


def _tc_out_body(leak_ref, pin_ref, s_ref, p0_ref, p1_ref, wrec_ref, o_ref):
    aggr = p0_ref[...] + p1_ref[...]
    dn = (((1,), (1,)), ((), ()))
    pre = pin_ref[...] + lax.dot_general(aggr, wrec_ref[...], dn,
                                         preferred_element_type=jnp.float32)
    lam = leak_ref[0, 0]
    o_ref[...] = lam * jnp.tanh(pre) + (1.0 - lam) * s_ref[...]


_BLK = 1000
_ROW_SPEC = pl.BlockSpec((_BLK, FEAT), lambda i: (i, 0))
_W_SPEC = pl.BlockSpec((FEAT, FEAT), lambda i: (0, 0))


@jax.jit
def _tc_in(x, W_in):
    return pl.pallas_call(
        _tc_in_body,
        grid=(N_NODES // _BLK,),
        in_specs=[_ROW_SPEC, _W_SPEC],
        out_specs=_ROW_SPEC,
        out_shape=jax.ShapeDtypeStruct((N_NODES, FEAT), jnp.float32),
    )(x, W_in)


@jax.jit
def _tc_out(leak, pin, s, p0, p1, W_rec):
    return pl.pallas_call(
        _tc_out_body,
        grid=(N_NODES // _BLK,),
        in_specs=[
            pl.BlockSpec(memory_space=pltpu.SMEM),
            _ROW_SPEC, _ROW_SPEC, _ROW_SPEC, _ROW_SPEC, _W_SPEC,
        ],
        out_specs=_ROW_SPEC,
        out_shape=jax.ShapeDtypeStruct((N_NODES, FEAT), jnp.float32),
    )(leak, pin, s, p0, p1, W_rec)


def kernel(edge_index, input, state, W_in, W_rec, leakage):
    src = edge_index[0].astype(jnp.int32)
    dst = edge_index[1].astype(jnp.int32)
    p0, p1 = _sc_scatter(src, dst, state)
    pin = _tc_in(input, W_in)
    leak2d = jnp.asarray(leakage, jnp.float32).reshape(1, 1)
    return _tc_out(leak2d, pin, state, p0, p1, W_rec)


# speedup vs baseline: 1.2218x; 1.0784x over previous
import jax
import jax.numpy as jnp
from jax import lax
from jax.experimental import pallas as pl
from jax.experimental.pallas import tpu as pltpu
from jax.experimental.pallas import tpu_sc as plsc

N_NODES = 10000
N_EDGES = 320000
FEAT = 128
NUM_CORES = 2
NUM_SUBCORES = 16
NUM_TILES = NUM_CORES * NUM_SUBCORES
GROUP = 40
N_PAD = 10240
EDGES_PER_TILE = N_EDGES // NUM_TILES
GROUPS = EDGES_PER_TILE // GROUP
ROWS_PER_TILE = N_PAD // NUM_SUBCORES
NROW_SLOTS = 4
NIDX_SLOTS = 8


def _sc_body(edges_hbm, state_hbm, out0, out1,
             idx_s, idx_d, rows, sem_i, sem_g, sem_s, shared):
    cid = lax.axis_index("c")
    sid = lax.axis_index("s")
    wid = cid * NUM_SUBCORES + sid

    zeros16 = jnp.zeros((16,), jnp.float32)

    def _zrow(r, _):
        def _zcol(j, _):
            rows[0, r, pl.ds(j * 16, 16)] = zeros16
            return 0
        return lax.fori_loop(0, FEAT // 16, _zcol, 0)

    lax.fori_loop(0, GROUP, _zrow, 0)

    row0 = sid * ROWS_PER_TILE
    for b in range(ROWS_PER_TILE // GROUP):
        pltpu.sync_copy(rows.at[0], shared.at[pl.ds(row0 + b * GROUP, GROUP)])
    plsc.subcore_barrier()

    ebase = wid * EDGES_PER_TILE

    def _fire_idx(g, slot):
        off = ebase + g * GROUP
        pltpu.async_copy(edges_hbm.at[pl.ds(off, GROUP)], idx_s.at[slot],
                         sem_i.at[slot])
        pltpu.async_copy(edges_hbm.at[pl.ds(N_EDGES + off, GROUP)],
                         idx_d.at[slot], sem_i.at[slot])

    def _drain_idx(slot):
        pltpu.make_async_copy(edges_hbm.at[pl.ds(0, GROUP)], idx_s.at[slot],
                              sem_i.at[slot]).wait()
        pltpu.make_async_copy(edges_hbm.at[pl.ds(0, GROUP)], idx_d.at[slot],
                              sem_i.at[slot]).wait()

    def _drain_rows(rslot, sem):
        pltpu.make_async_copy(state_hbm.at[pl.ds(0, GROUP)], rows.at[rslot],
                              sem.at[rslot]).wait()

    def _fire_gather(islot, rslot):
        pltpu.async_copy(state_hbm.at[idx_s.at[islot]], rows.at[rslot],
                         sem_g.at[rslot])

    for g0 in range(NIDX_SLOTS - 1):
        _fire_idx(g0, g0)
    for g0 in range(NROW_SLOTS - 1):
        _drain_idx(g0)
        _fire_gather(g0, g0)

    def _iter(i, _):
        for j in range(NIDX_SLOTS):
            g = i * NIDX_SLOTS + j
            rslot = j % NROW_SLOTS
            fslot = (j + NROW_SLOTS - 1) % NROW_SLOTS

            @pl.when(g + NROW_SLOTS - 1 < GROUPS)
            def _():
                @pl.when(g >= 1)
                def _():
                    _drain_rows(fslot, sem_s)
                _drain_idx((j + NROW_SLOTS - 1) % NIDX_SLOTS)
                _fire_gather((j + NROW_SLOTS - 1) % NIDX_SLOTS, fslot)

            @pl.when(g + NIDX_SLOTS - 1 < GROUPS)
            def _():
                _fire_idx(g + NIDX_SLOTS - 1, (j + NIDX_SLOTS - 1) % NIDX_SLOTS)

            _drain_rows(rslot, sem_g)
            pltpu.async_copy(rows.at[rslot], shared.at[idx_d.at[j]],
                             sem_s.at[rslot], add=True)
        return 0

    lax.fori_loop(0, GROUPS // NIDX_SLOTS, _iter, 0)
    ntail = GROUPS - (GROUPS // NIDX_SLOTS) * NIDX_SLOTS
    for t in range(ntail):
        _drain_rows(t % NROW_SLOTS, sem_g)
        pltpu.async_copy(rows.at[t % NROW_SLOTS], shared.at[idx_d.at[t]],
                         sem_s.at[t % NROW_SLOTS], add=True)
    for t in range(NROW_SLOTS):
        _drain_rows((ntail + t) % NROW_SLOTS, sem_s)
    plsc.subcore_barrier()

    @pl.when(cid == 0)
    def _():
        pltpu.sync_copy(shared.at[pl.ds(row0, ROWS_PER_TILE)],
                        out0.at[pl.ds(row0, ROWS_PER_TILE)])

    @pl.when(cid == 1)
    def _():
        pltpu.sync_copy(shared.at[pl.ds(row0, ROWS_PER_TILE)],
                        out1.at[pl.ds(row0, ROWS_PER_TILE)])


@jax.jit
def _sc_scatter(edges, state):
    mesh = plsc.VectorSubcoreMesh(core_axis_name="c", subcore_axis_name="s")
    f = pl.kernel(
        _sc_body,
        out_type=[jax.ShapeDtypeStruct((N_PAD, FEAT), jnp.float32),
                  jax.ShapeDtypeStruct((N_PAD, FEAT), jnp.float32)],
        mesh=mesh,
        scratch_types=[
            pltpu.VMEM((NIDX_SLOTS, GROUP), jnp.int32),
            pltpu.VMEM((NIDX_SLOTS, GROUP), jnp.int32),
            pltpu.VMEM((NROW_SLOTS, GROUP, FEAT), jnp.float32),
            pltpu.SemaphoreType.DMA((NIDX_SLOTS,)),
            pltpu.SemaphoreType.DMA((NROW_SLOTS,)),
            pltpu.SemaphoreType.DMA((NROW_SLOTS,)),
            pltpu.VMEM_SHARED((N_PAD, FEAT), jnp.float32),
        ],
    )
    return f(edges, state)


def _tc_body(leak_ref, x_ref, s_ref, p0_ref, p1_ref, win_ref, wrec_ref, o_ref):
    aggr = p0_ref[...] + p1_ref[...]
    dn = (((1,), (1,)), ((), ()))
    pre = lax.dot_general(x_ref[...], win_ref[...], dn,
                          preferred_element_type=jnp.float32)
    pre = pre + lax.dot_general(aggr, wrec_ref[...], dn,
                                preferred_element_type=jnp.float32)
    lam = leak_ref[0, 0]
    o_ref[...] = lam * jnp.tanh(pre) + (1.0 - lam) * s_ref[...]


_BLK = 1000
_ROW_SPEC = pl.BlockSpec((_BLK, FEAT), lambda i: (i, 0))
_W_SPEC = pl.BlockSpec((FEAT, FEAT), lambda i: (0, 0))


@jax.jit
def _tc_dense(leak, x, s, p0, p1, W_in, W_rec):
    return pl.pallas_call(
        _tc_body,
        grid=(N_NODES // _BLK,),
        in_specs=[
            pl.BlockSpec(memory_space=pltpu.SMEM),
            _ROW_SPEC, _ROW_SPEC, _ROW_SPEC, _ROW_SPEC, _W_SPEC, _W_SPEC,
        ],
        out_specs=_ROW_SPEC,
        out_shape=jax.ShapeDtypeStruct((N_NODES, FEAT), jnp.float32),
    )(leak, x, s, p0, p1, W_in, W_rec)


def kernel(edge_index, input, state, W_in, W_rec, leakage):
    edges = edge_index.astype(jnp.int32).reshape(-1)
    p0, p1 = _sc_scatter(edges, state)
    leak2d = jnp.asarray(leakage, jnp.float32).reshape(1, 1)
    return _tc_dense(leak2d, input, state, p0, p1, W_in, W_rec)


# speedup vs baseline: 1.2327x; 1.0089x over previous
import jax
import jax.numpy as jnp
from jax import lax
from jax.experimental import pallas as pl
from jax.experimental.pallas import tpu as pltpu
from jax.experimental.pallas import tpu_sc as plsc

N_NODES = 10000
N_EDGES = 320000
FEAT = 128
NUM_CORES = 2
NUM_SUBCORES = 16
NUM_TILES = NUM_CORES * NUM_SUBCORES
GROUP = 40
N_PAD = 10240
EDGES_PER_TILE = N_EDGES // NUM_TILES
GROUPS = EDGES_PER_TILE // GROUP
ROWS_PER_TILE = N_PAD // NUM_SUBCORES
NROW_SLOTS = 4
NIDX_SLOTS = 8


def _sc_body(edges_hbm, state_hbm, out0, out1,
             idx_s, idx_d, rows, sem_i, sem_g, sem_s, shared):
    cid = lax.axis_index("c")
    sid = lax.axis_index("s")
    wid = cid * NUM_SUBCORES + sid

    row0 = sid * ROWS_PER_TILE
    ebase = wid * EDGES_PER_TILE

    def _fire_idx(g, slot):
        off = ebase + g * GROUP
        pltpu.async_copy(edges_hbm.at[pl.ds(off, GROUP)], idx_s.at[slot],
                         sem_i.at[slot])
        pltpu.async_copy(edges_hbm.at[pl.ds(N_EDGES + off, GROUP)],
                         idx_d.at[slot], sem_i.at[slot])

    def _drain_idx(slot):
        pltpu.make_async_copy(edges_hbm.at[pl.ds(0, GROUP)], idx_s.at[slot],
                              sem_i.at[slot]).wait()
        pltpu.make_async_copy(edges_hbm.at[pl.ds(0, GROUP)], idx_d.at[slot],
                              sem_i.at[slot]).wait()

    def _drain_rows(rslot, sem):
        pltpu.make_async_copy(state_hbm.at[pl.ds(0, GROUP)], rows.at[rslot],
                              sem.at[rslot]).wait()

    def _fire_gather(islot, rslot):
        pltpu.async_copy(state_hbm.at[idx_s.at[islot]], rows.at[rslot],
                         sem_g.at[rslot])

    for g0 in range(NIDX_SLOTS - 1):
        _fire_idx(g0, g0)
    for g0 in range(NROW_SLOTS - 1):
        _drain_idx(g0)
        _fire_gather(g0, g0)

    zeros16 = jnp.zeros((16,), jnp.float32)
    zslot = NROW_SLOTS - 1

    def _zrow(r, _):
        def _zcol(j, _):
            rows[zslot, r, pl.ds(j * 16, 16)] = zeros16
            return 0
        return lax.fori_loop(0, FEAT // 16, _zcol, 0)

    lax.fori_loop(0, GROUP, _zrow, 0)
    for b in range(ROWS_PER_TILE // GROUP):
        pltpu.sync_copy(rows.at[zslot],
                        shared.at[pl.ds(row0 + b * GROUP, GROUP)])
    plsc.subcore_barrier()

    def _iter(i, _):
        for j in range(NIDX_SLOTS):
            g = i * NIDX_SLOTS + j
            rslot = j % NROW_SLOTS
            fslot = (j + NROW_SLOTS - 1) % NROW_SLOTS

            @pl.when(g + NROW_SLOTS - 1 < GROUPS)
            def _():
                @pl.when(g >= 1)
                def _():
                    _drain_rows(fslot, sem_s)
                _drain_idx((j + NROW_SLOTS - 1) % NIDX_SLOTS)
                _fire_gather((j + NROW_SLOTS - 1) % NIDX_SLOTS, fslot)

            @pl.when(g + NIDX_SLOTS - 1 < GROUPS)
            def _():
                _fire_idx(g + NIDX_SLOTS - 1, (j + NIDX_SLOTS - 1) % NIDX_SLOTS)

            _drain_rows(rslot, sem_g)
            pltpu.async_copy(rows.at[rslot], shared.at[idx_d.at[j]],
                             sem_s.at[rslot], add=True)
        return 0

    lax.fori_loop(0, GROUPS // NIDX_SLOTS, _iter, 0)
    ntail = GROUPS - (GROUPS // NIDX_SLOTS) * NIDX_SLOTS
    for t in range(ntail):
        _drain_rows(t % NROW_SLOTS, sem_g)
        pltpu.async_copy(rows.at[t % NROW_SLOTS], shared.at[idx_d.at[t]],
                         sem_s.at[t % NROW_SLOTS], add=True)
    for t in range(NROW_SLOTS):
        _drain_rows((ntail + t) % NROW_SLOTS, sem_s)
    plsc.subcore_barrier()

    @pl.when(cid == 0)
    def _():
        pltpu.sync_copy(shared.at[pl.ds(row0, ROWS_PER_TILE)],
                        out0.at[pl.ds(row0, ROWS_PER_TILE)])

    @pl.when(cid == 1)
    def _():
        pltpu.sync_copy(shared.at[pl.ds(row0, ROWS_PER_TILE)],
                        out1.at[pl.ds(row0, ROWS_PER_TILE)])


@jax.jit
def _sc_scatter(edges, state):
    mesh = plsc.VectorSubcoreMesh(core_axis_name="c", subcore_axis_name="s")
    f = pl.kernel(
        _sc_body,
        out_type=[jax.ShapeDtypeStruct((N_PAD, FEAT), jnp.float32),
                  jax.ShapeDtypeStruct((N_PAD, FEAT), jnp.float32)],
        mesh=mesh,
        scratch_types=[
            pltpu.VMEM((NIDX_SLOTS, GROUP), jnp.int32),
            pltpu.VMEM((NIDX_SLOTS, GROUP), jnp.int32),
            pltpu.VMEM((NROW_SLOTS, GROUP, FEAT), jnp.float32),
            pltpu.SemaphoreType.DMA((NIDX_SLOTS,)),
            pltpu.SemaphoreType.DMA((NROW_SLOTS,)),
            pltpu.SemaphoreType.DMA((NROW_SLOTS,)),
            pltpu.VMEM_SHARED((N_PAD, FEAT), jnp.float32),
        ],
    )
    return f(edges, state)


def _tc_body(leak_ref, x_ref, s_ref, p0_ref, p1_ref, win_ref, wrec_ref, o_ref):
    aggr = p0_ref[...] + p1_ref[...]
    dn = (((1,), (1,)), ((), ()))
    pre = lax.dot_general(x_ref[...], win_ref[...], dn,
                          preferred_element_type=jnp.float32)
    pre = pre + lax.dot_general(aggr, wrec_ref[...], dn,
                                preferred_element_type=jnp.float32)
    lam = leak_ref[0, 0]
    o_ref[...] = lam * jnp.tanh(pre) + (1.0 - lam) * s_ref[...]


_BLK = 1000
_ROW_SPEC = pl.BlockSpec((_BLK, FEAT), lambda i: (i, 0))
_W_SPEC = pl.BlockSpec((FEAT, FEAT), lambda i: (0, 0))


@jax.jit
def _tc_dense(leak, x, s, p0, p1, W_in, W_rec):
    return pl.pallas_call(
        _tc_body,
        grid=(N_NODES // _BLK,),
        in_specs=[
            pl.BlockSpec(memory_space=pltpu.SMEM),
            _ROW_SPEC, _ROW_SPEC, _ROW_SPEC, _ROW_SPEC, _W_SPEC, _W_SPEC,
        ],
        out_specs=_ROW_SPEC,
        out_shape=jax.ShapeDtypeStruct((N_NODES, FEAT), jnp.float32),
    )(leak, x, s, p0, p1, W_in, W_rec)


def kernel(edge_index, input, state, W_in, W_rec, leakage):
    edges = edge_index.astype(jnp.int32).reshape(-1)
    p0, p1 = _sc_scatter(edges, state)
    leak2d = jnp.asarray(leakage, jnp.float32).reshape(1, 1)
    return _tc_dense(leak2d, input, state, p0, p1, W_in, W_rec)
